# Initial kernel scaffold; baseline (speedup 1.0000x reference)
import jax
import jax.numpy as jnp
from jax.experimental import pallas as pl


def kernel(input):
    raise NotImplementedError("write your pallas kernel here")


# speedup vs baseline: 7.9070x; 7.9070x over previous
import functools

import jax
import jax.numpy as jnp
from jax.experimental import pallas as pl

NUM_DOMAINS = 4
BATCH = 1024
K = 20
FEAT = 1024

_NEG = float("-inf")


def _norm_body(x_ref, o_ref):
    x = x_ref[...]
    s = jnp.sum(x * x, axis=1, keepdims=True)
    n = jnp.sqrt(s)
    o_ref[...] = x / jnp.maximum(n, 1e-12)


def _main_body(a_ref, b_ref, adj_ref, idx_ref, *, nb, batch, k, klanes):
    i = pl.program_id(0)
    j = pl.program_id(1)
    last = nb - 1
    compute = jnp.logical_or(i == j, jnp.logical_and(j == last, i < last))

    @pl.when(compute)
    def _():
        a = a_ref[...].astype(jnp.bfloat16)
        b = b_ref[...].astype(jnp.bfloat16)
        sim = jax.lax.dot_general(
            a, b, (((1,), (1,)), ((), ())),
            preferred_element_type=jnp.float32,
        )
        lane = jax.lax.broadcasted_iota(jnp.int32, (batch, batch), 1)
        lanek = jax.lax.broadcasted_iota(jnp.int32, (batch, klanes), 1)
        work = sim
        idx_acc = jnp.zeros((batch, klanes), jnp.int32)
        theta = None
        for t in range(k):
            m = jnp.max(work, axis=1, keepdims=True)
            eq = work == m
            col = jnp.min(jnp.where(eq, lane, batch), axis=1, keepdims=True)
            idx_acc = jnp.where(lanek == t, col, idx_acc)
            work = jnp.where(lane == col, _NEG, work)
            theta = m
        adj_ref[...] = jnp.where(sim >= theta, sim, 0.0)

        @pl.when(i == j)
        def _():
            idx_ref[0] = idx_acc

    @pl.when(jnp.logical_not(compute))
    def _():
        adj_ref[...] = jnp.zeros((batch, batch), jnp.float32)


def _build(nd, batch, feat, k):
    nb = nd + 1
    n = nb * batch
    klanes = 128

    norm = pl.pallas_call(
        _norm_body,
        grid=(nb,),
        in_specs=[pl.BlockSpec((batch, feat), lambda g: (g, 0))],
        out_specs=pl.BlockSpec((batch, feat), lambda g: (g, 0)),
        out_shape=jax.ShapeDtypeStruct((n, feat), jnp.float32),
    )

    main = pl.pallas_call(
        functools.partial(_main_body, nb=nb, batch=batch, k=k, klanes=klanes),
        grid=(nb, nb),
        in_specs=[
            pl.BlockSpec((batch, feat), lambda i, j: (i, 0)),
            pl.BlockSpec((batch, feat), lambda i, j: (j, 0)),
        ],
        out_specs=[
            pl.BlockSpec((batch, batch), lambda i, j: (i, j)),
            pl.BlockSpec((1, batch, klanes), lambda i, j: (i, 0, 0)),
        ],
        out_shape=[
            jax.ShapeDtypeStruct((n, n), jnp.float32),
            jax.ShapeDtypeStruct((nb, batch, klanes), jnp.int32),
        ],
    )

    def fn(x):
        xn = norm(x)
        adj, idxp = main(xn, xn)
        return adj, idxp[:, :, :k]

    return fn


_kernel_impl = _build(NUM_DOMAINS, BATCH, FEAT, K)


def kernel(input):
    return _kernel_impl(input)
